# Initial kernel scaffold; baseline (speedup 1.0000x reference)
import jax
import jax.numpy as jnp
from jax.experimental import pallas as pl


def kernel(x, edge_index, W1, b1, W2, b2, W_out, b_out):
    raise NotImplementedError("write your pallas kernel here")


# speedup vs baseline: 14.6163x; 14.6163x over previous
import functools

import jax
import jax.numpy as jnp
from jax import lax
from jax.experimental import pallas as pl
from jax.experimental.pallas import tpu as pltpu
from jax.experimental.pallas import tpu_sc as plsc

N = 10000
D = 128
E = 320000

CH = 128
NCH = E // CH
NTILES = 32
FULL = NCH // NTILES
REM = NCH - FULL * NTILES
NSUB = 16
ROWS_PT = 624
TAIL = N - ROWS_PT * NSUB
TAIL_OFF = ROWS_PT * NSUB

BM = 1000
GRID = N // BM

_mesh = plsc.VectorSubcoreMesh(core_axis_name="c", subcore_axis_name="s")


def _sc_degree(dst, ones_rows, zeros_nd):
    @functools.partial(
        pl.kernel,
        out_type=jax.ShapeDtypeStruct((2, N, D), jnp.float32),
        mesh=_mesh,
        scratch_types=[
            pltpu.VMEM((CH,), jnp.int32),
            pltpu.VMEM((CH, D), jnp.float32),
            pltpu.VMEM_SHARED((N, D), jnp.float32),
        ],
    )
    def deg_kernel(dst_hbm, ones_hbm, zeros_hbm, out_hbm, idx_v, ones_v, acc_sh):
        c = lax.axis_index("c")
        s = lax.axis_index("s")
        t = c * NSUB + s
        pltpu.sync_copy(ones_hbm, ones_v)
        pltpu.sync_copy(
            zeros_hbm.at[pl.ds(s * ROWS_PT, ROWS_PT)],
            acc_sh.at[pl.ds(s * ROWS_PT, ROWS_PT)],
        )

        @pl.when(s == 0)
        def _():
            pltpu.sync_copy(
                zeros_hbm.at[pl.ds(TAIL_OFF, TAIL)],
                acc_sh.at[pl.ds(TAIL_OFF, TAIL)],
            )

        plsc.subcore_barrier()

        def do_chunk(j):
            base = j * CH
            pltpu.sync_copy(dst_hbm.at[pl.ds(base, CH)], idx_v)
            pltpu.sync_copy(ones_v, acc_sh.at[idx_v], add=True)

        @pl.loop(0, FULL)
        def _(i):
            do_chunk(t + i * NTILES)

        @pl.when(t < REM)
        def _():
            do_chunk(FULL * NTILES + t)

        plsc.subcore_barrier()
        pltpu.sync_copy(
            acc_sh.at[pl.ds(s * ROWS_PT, ROWS_PT)],
            out_hbm.at[c, pl.ds(s * ROWS_PT, ROWS_PT)],
        )

        @pl.when(s == 0)
        def _():
            pltpu.sync_copy(
                acc_sh.at[pl.ds(TAIL_OFF, TAIL)],
                out_hbm.at[c, pl.ds(TAIL_OFF, TAIL)],
            )

    return deg_kernel(dst, ones_rows, zeros_nd)


def _sc_gather_scatter(hp, src, dst, zeros_nd):
    @functools.partial(
        pl.kernel,
        out_type=jax.ShapeDtypeStruct((2, N, D), jnp.float32),
        mesh=_mesh,
        scratch_types=[
            pltpu.VMEM((CH,), jnp.int32),
            pltpu.VMEM((CH,), jnp.int32),
            pltpu.VMEM((CH, D), jnp.float32),
            pltpu.VMEM_SHARED((N, D), jnp.float32),
            pltpu.SemaphoreType.DMA,
        ],
    )
    def gs_kernel(hp_hbm, src_hbm, dst_hbm, zeros_hbm, out_hbm,
                  src_v, dst_v, rows_v, acc_sh, sem):
        c = lax.axis_index("c")
        s = lax.axis_index("s")
        t = c * NSUB + s
        pltpu.sync_copy(
            zeros_hbm.at[pl.ds(s * ROWS_PT, ROWS_PT)],
            acc_sh.at[pl.ds(s * ROWS_PT, ROWS_PT)],
        )

        @pl.when(s == 0)
        def _():
            pltpu.sync_copy(
                zeros_hbm.at[pl.ds(TAIL_OFF, TAIL)],
                acc_sh.at[pl.ds(TAIL_OFF, TAIL)],
            )

        plsc.subcore_barrier()

        def do_chunk(j):
            base = j * CH
            pltpu.sync_copy(src_hbm.at[pl.ds(base, CH)], src_v)
            pltpu.sync_copy(dst_hbm.at[pl.ds(base, CH)], dst_v)
            pltpu.async_copy(hp_hbm.at[src_v], rows_v, sem).wait()
            pltpu.sync_copy(rows_v, acc_sh.at[dst_v], add=True)

        @pl.loop(0, FULL)
        def _(i):
            do_chunk(t + i * NTILES)

        @pl.when(t < REM)
        def _():
            do_chunk(FULL * NTILES + t)

        plsc.subcore_barrier()
        pltpu.sync_copy(
            acc_sh.at[pl.ds(s * ROWS_PT, ROWS_PT)],
            out_hbm.at[c, pl.ds(s * ROWS_PT, ROWS_PT)],
        )

        @pl.when(s == 0)
        def _():
            pltpu.sync_copy(
                acc_sh.at[pl.ds(TAIL_OFF, TAIL)],
                out_hbm.at[c, pl.ds(TAIL_OFF, TAIL)],
            )

    return gs_kernel(hp, src, dst, zeros_nd)


_DN = (((1,), (0,)), ((), ()))


def _tc_first(x, W1, degp):
    def body(x_ref, w_ref, deg_ref, hp_ref, dinv_ref):
        deg = deg_ref[0, :, 0:1] + deg_ref[1, :, 0:1] + 1.0
        dinv = jnp.broadcast_to(lax.rsqrt(deg), (BM, D))
        h = lax.dot_general(x_ref[...], w_ref[...], _DN,
                            preferred_element_type=jnp.float32,
                            precision=lax.Precision.HIGHEST)
        hp_ref[...] = h * dinv
        dinv_ref[...] = dinv

    return pl.pallas_call(
        body,
        grid=(GRID,),
        in_specs=[
            pl.BlockSpec((BM, D), lambda i: (i, 0)),
            pl.BlockSpec((D, D), lambda i: (0, 0)),
            pl.BlockSpec((2, BM, D), lambda i: (0, i, 0)),
        ],
        out_specs=[
            pl.BlockSpec((BM, D), lambda i: (i, 0)),
            pl.BlockSpec((BM, D), lambda i: (i, 0)),
        ],
        out_shape=[
            jax.ShapeDtypeStruct((N, D), jnp.float32),
            jax.ShapeDtypeStruct((N, D), jnp.float32),
        ],
    )(x, W1, degp)


def _tc_mid(agg, hp, dinv, b1, W2):
    def body(agg_ref, hp_ref, dinv_ref, b_ref, w_ref, out_ref):
        total = agg_ref[0] + agg_ref[1] + hp_ref[...]
        out1 = jnp.maximum(dinv_ref[...] * total + b_ref[...], 0.0)
        h2 = lax.dot_general(out1, w_ref[...], _DN,
                             preferred_element_type=jnp.float32,
                             precision=lax.Precision.HIGHEST)
        out_ref[...] = dinv_ref[...] * h2

    return pl.pallas_call(
        body,
        grid=(GRID,),
        in_specs=[
            pl.BlockSpec((2, BM, D), lambda i: (0, i, 0)),
            pl.BlockSpec((BM, D), lambda i: (i, 0)),
            pl.BlockSpec((BM, D), lambda i: (i, 0)),
            pl.BlockSpec((1, D), lambda i: (0, 0)),
            pl.BlockSpec((D, D), lambda i: (0, 0)),
        ],
        out_specs=pl.BlockSpec((BM, D), lambda i: (i, 0)),
        out_shape=jax.ShapeDtypeStruct((N, D), jnp.float32),
    )(agg, hp, dinv, b1, W2)


def _tc_last(agg, hp, dinv, b2, W_out, b_out):
    def body(agg_ref, hp_ref, dinv_ref, b2_ref, w_ref, bo_ref, out_ref):
        total = agg_ref[0] + agg_ref[1] + hp_ref[...]
        out2 = dinv_ref[...] * total + b2_ref[...]
        h = lax.dot_general(out2, w_ref[...], _DN,
                            preferred_element_type=jnp.float32,
                            precision=lax.Precision.HIGHEST)
        out_ref[...] = h + bo_ref[...]

    return pl.pallas_call(
        body,
        grid=(GRID,),
        in_specs=[
            pl.BlockSpec((2, BM, D), lambda i: (0, i, 0)),
            pl.BlockSpec((BM, D), lambda i: (i, 0)),
            pl.BlockSpec((BM, D), lambda i: (i, 0)),
            pl.BlockSpec((1, D), lambda i: (0, 0)),
            pl.BlockSpec((D, D), lambda i: (0, 0)),
            pl.BlockSpec((1, D), lambda i: (0, 0)),
        ],
        out_specs=pl.BlockSpec((BM, D), lambda i: (i, 0)),
        out_shape=jax.ShapeDtypeStruct((N, D), jnp.float32),
    )(agg, hp, dinv, b2, W_out, b_out)


def kernel(x, edge_index, W1, b1, W2, b2, W_out, b_out):
    src = edge_index[0]
    dst = edge_index[1]
    ones_rows = jnp.ones((CH, D), jnp.float32)
    zeros_nd = jnp.zeros((N, D), jnp.float32)

    degp = _sc_degree(dst, ones_rows, zeros_nd)
    hp1, dinv = _tc_first(x, W1, degp)
    agg1 = _sc_gather_scatter(hp1, src, dst, zeros_nd)
    hp2 = _tc_mid(agg1, hp1, dinv, b1.reshape(1, D), W2)
    agg2 = _sc_gather_scatter(hp2, src, dst, zeros_nd)
    return _tc_last(agg2, hp2, dinv, b2.reshape(1, D), W_out, b_out.reshape(1, D))
